# baseline (speedup 1.0000x reference)
import functools

import jax
import jax.numpy as jnp
from jax import lax
from jax.experimental import pallas as pl
from jax.experimental.pallas import tpu as pltpu
from jax.experimental.pallas import tpu_sc as plsc


def _tc_body(p_ref, t_ref, out_ref, tok_ref, *, RC, R, L, D, C):
    c = pl.program_id(1)
    p = p_ref[0]
    t = t_ref[0]
    d = p - t
    e = d * d
    ones_d = jnp.ones((D,), jnp.float32)
    tokc = lax.dot_general(
        e, ones_d, (((2,), (0,)), ((), ())),
        preferred_element_type=jnp.float32,
        precision=lax.Precision.HIGHEST) * (1.0 / D)
    tok_ref[pl.ds(c * RC, RC), :] = tokc

    @pl.when(c == C - 1)
    def _():
        tok = tok_ref[...]
        io_r = lax.broadcasted_iota(jnp.int32, (L, L), 0)
        io_c = lax.broadcasted_iota(jnp.int32, (L, L), 1)
        upper_incl = (io_r <= io_c).astype(jnp.float32)
        within = lax.dot_general(
            tok, upper_incl, (((1,), (0,)), ((), ())),
            preferred_element_type=jnp.float32,
            precision=lax.Precision.HIGHEST)
        ones_l = jnp.ones((L, 1), jnp.float32)
        rowtot = lax.dot_general(
            tok, ones_l, (((1,), (0,)), ((), ())),
            preferred_element_type=jnp.float32,
            precision=lax.Precision.HIGHEST)
        jo_r = lax.broadcasted_iota(jnp.int32, (R, R), 0)
        jo_c = lax.broadcasted_iota(jnp.int32, (R, R), 1)
        strict_lower = (jo_c < jo_r).astype(jnp.float32)
        offs = lax.dot_general(
            strict_lower, rowtot, (((1,), (0,)), ((), ())),
            preferred_element_type=jnp.float32,
            precision=lax.Precision.HIGHEST)
        out_ref[0] = within + offs


def _sc_body(csum_hbm, idx_hbm, it_hbm, out_hbm,
             csum_v, idx_v, it_v, acc_v, out_v, all_v, shared,
             *, B, N, G, T, NS):
    s = lax.axis_index("s")
    pltpu.sync_copy(csum_hbm.at[s], csum_v)
    pltpu.sync_copy(idx_hbm.at[s], idx_v)
    pltpu.sync_copy(it_hbm.at[pl.ds(s * (G * T), G * T)], it_v)

    lanes = lax.iota(jnp.int32, 16)
    zeros16 = jnp.zeros((16,), jnp.float32)
    acc_v[pl.ds(0, 16)] = zeros16
    acc_v[pl.ds(16, 16)] = zeros16
    acc_v[pl.ds(32, 16)] = zeros16
    acc_v[pl.ds(48, 16)] = zeros16
    acc_v[pl.ds(64, 16)] = zeros16
    acc_v[pl.ds(80, 16)] = zeros16
    acc_v[pl.ds(96, 16)] = zeros16
    acc_v[pl.ds(112, 16)] = zeros16

    for j in range(G // 16):
        g = j * 16 + lanes
        starts = plsc.load_gather(idx_v, [g])
        ends = plsc.load_gather(idx_v, [g + 1])
        cs_e = plsc.load_gather(csum_v, [jnp.maximum(ends - 1, 0)])
        cs_e = jnp.where(ends > 0, cs_e, 0.0)
        cs_s = plsc.load_gather(csum_v, [jnp.maximum(starts - 1, 0)])
        cs_s = jnp.where(starts > 0, cs_s, 0.0)
        cnt = (ends - starts).astype(jnp.float32)
        gerr = (cs_e - cs_s) / jnp.maximum(cnt, 1.0)

        base = g * T
        m = plsc.load_gather(it_v, [base])
        am = jnp.zeros((16,), jnp.int32)
        for tt in range(1, T):
            col = plsc.load_gather(it_v, [base + tt])
            better = col > m
            am = jnp.where(better, tt, am)
            m = jnp.where(better, col, m)

        plsc.addupdate_scatter(acc_v, [am], gerr)
        plsc.addupdate_scatter(acc_v, [am + 16], jnp.ones((16,), jnp.float32))

    pltpu.sync_copy(acc_v.at[pl.ds(0, 2 * T)], shared.at[pl.ds(s * (2 * T), 2 * T)])
    plsc.subcore_barrier()

    @pl.when(s == 0)
    def _():
        pltpu.sync_copy(shared, all_v)
        ts = jnp.zeros((16,), jnp.float32)
        tc = jnp.zeros((16,), jnp.float32)
        for i in range(NS):
            ts = ts + all_v[pl.ds(i * (2 * T), 16)]
            tc = tc + all_v[pl.ds(i * (2 * T) + 16, 16)]
        per = jnp.where(tc > 0, ts / jnp.maximum(tc, 1.0), 0.0)
        out_v[...] = per
        pltpu.sync_copy(out_v, out_hbm)


def kernel(pred, target, indices, indices_type, type_names):
    B, N, D = pred.shape
    G = indices.shape[1] - 1
    T = indices_type.shape[2]
    R, L = 32, 128
    C = 4
    RC = R // C

    pred4 = pred.reshape(B, R, L, D)
    target4 = target.reshape(B, R, L, D)
    csum = pl.pallas_call(
        functools.partial(_tc_body, RC=RC, R=R, L=L, D=D, C=C),
        grid=(B, C),
        in_specs=[
            pl.BlockSpec((1, RC, L, D), lambda b, c: (b, c, 0, 0)),
            pl.BlockSpec((1, RC, L, D), lambda b, c: (b, c, 0, 0)),
        ],
        out_specs=pl.BlockSpec((1, R, L), lambda b, c: (b, 0, 0)),
        out_shape=jax.ShapeDtypeStruct((B, R, L), jnp.float32),
        scratch_shapes=[pltpu.VMEM((R, L), jnp.float32)],
    )(pred4, target4)
    csum2 = csum.reshape(B, N)

    IDXP = 128
    idx_pad = jnp.pad(indices, ((0, 0), (0, IDXP - (G + 1))))
    it_flat = indices_type.reshape(B * G * T)

    mesh = plsc.VectorSubcoreMesh(core_axis_name="c", subcore_axis_name="s")
    NS = 16
    sc_call = functools.partial(
        pl.kernel,
        out_type=jax.ShapeDtypeStruct((T,), jnp.float32),
        mesh=mesh,
        compiler_params=pltpu.CompilerParams(needs_layout_passes=False),
        scratch_types=[
            pltpu.VMEM((N,), jnp.float32),
            pltpu.VMEM((IDXP,), jnp.int32),
            pltpu.VMEM((G * T,), jnp.float32),
            pltpu.VMEM((128,), jnp.float32),
            pltpu.VMEM((T,), jnp.float32),
            pltpu.VMEM((NS * 2 * T,), jnp.float32),
            pltpu.VMEM_SHARED((NS * 2 * T,), jnp.float32),
        ],
    )(functools.partial(_sc_body, B=B, N=N, G=G, T=T, NS=NS))
    out = sc_call(csum2, idx_pad, it_flat)
    return out
